# baseline (speedup 1.0000x reference)
import functools

import jax
import jax.numpy as jnp
from jax import lax
from jax.experimental import pallas as pl
from jax.experimental.pallas import tpu as pltpu
from jax.experimental.pallas import tpu_sc as plsc

_NW = 32
_CH = 128

_INV2PI = 0.15915494309189535
_TWOPI = 6.283185307179586
_S1 = 9.9998459345e-01
_S3 = -1.6663259377e-01
_S5 = 8.3123882797e-03
_S7 = -1.9316269889e-04
_S9 = 2.1732569601e-06


def _psin(x):
    n = jnp.floor(x * _INV2PI + 0.5)
    r = x - n * _TWOPI
    r2 = r * r
    return r * (_S1 + r2 * (_S3 + r2 * (_S5 + r2 * (_S7 + r2 * _S9))))


def _sc_transpose(tt):
    d, v = tt.shape
    pairs = v // 2
    full = v // 256
    tail = (v - full * 256) // 2
    per_w = full // _NW
    extra = full - per_w * _NW
    mesh = plsc.VectorSubcoreMesh(core_axis_name="c", subcore_axis_name="s")

    @functools.partial(
        pl.kernel,
        mesh=mesh,
        out_type=jax.ShapeDtypeStruct((pairs, 2 * d), jnp.float32),
        compiler_params=pltpu.CompilerParams(use_tc_tiling_on_sc=True,
                                             needs_layout_passes=False),
        scratch_types=[
            pltpu.VMEM((d, 256), jnp.float32),
            pltpu.VMEM((d, 256), jnp.float32),
            pltpu.VMEM((128, 2 * d), jnp.float32),
            pltpu.VMEM((128, 2 * d), jnp.float32),
            pltpu.SemaphoreType.DMA,
            pltpu.SemaphoreType.DMA,
            pltpu.SemaphoreType.DMA,
            pltpu.SemaphoreType.DMA,
        ],
    )
    def k(tt_hbm, tail_hbm, out_hbm, sla, slb, outa, outb, ssa, ssb, swa, swb):
        wid = lax.axis_index("s") * 2 + lax.axis_index("c")
        iota = lax.iota(jnp.int32, 16)
        rowv = [iota + 16 * qq for qq in range(4)]

        def chunk_of(i):
            return i * _NW + wid

        def fire_slab(c, slab, sem):
            pltpu.async_copy(tt_hbm.at[:, pl.ds(c * 256, 256)], slab, sem)

        def transpose_slab(slab, outb_, nu):
            def urow(u, carry):
                for half in range(2):
                    colv = jnp.zeros((16,), jnp.int32) + (2 * u + half)
                    for qq in range(4):
                        vals = plsc.load_gather(slab, [rowv[qq], colv])
                        outb_[u, pl.ds(half * d + 16 * qq, 16)] = vals
                return carry
            lax.fori_loop(0, nu, urow, 0)

        def stage(g, slab, outb_, sem_s, sem_w, off):
            c = chunk_of(2 * g + off)
            pltpu.make_async_copy(tt_hbm.at[:, pl.ds(0, 256)], slab, sem_s).wait()

            @pl.when(g > 0)
            def _():
                pltpu.make_async_copy(outb_, out_hbm.at[pl.ds(0, 128)], sem_w).wait()

            transpose_slab(slab, outb_, 128)
            pltpu.async_copy(outb_, out_hbm.at[pl.ds(c * 128, 128)], sem_w)

            nxt = chunk_of(2 * (g + 1) + off)

            @pl.when(nxt < full)
            def _():
                fire_slab(nxt, slab, sem_s)

        fire_slab(chunk_of(0), sla, ssa)
        fire_slab(chunk_of(1), slb, ssb)

        def body(g, carry):
            stage(g, sla, outa, ssa, swa, 0)
            stage(g, slb, outb, ssb, swb, 1)
            return carry

        lax.fori_loop(0, per_w // 2, body, 0)

        @pl.when(wid < extra)
        def _():
            c = chunk_of(per_w)
            pltpu.make_async_copy(tt_hbm.at[:, pl.ds(0, 256)], sla, ssa).wait()
            pltpu.make_async_copy(outa, out_hbm.at[pl.ds(0, 128)], swa).wait()
            transpose_slab(sla, outa, 128)
            pltpu.async_copy(outa, out_hbm.at[pl.ds(c * 128, 128)], swa)

        @pl.when(wid == extra)
        def _():
            pltpu.make_async_copy(outb, out_hbm.at[pl.ds(0, 128)], swb).wait()
            pltpu.sync_copy(tail_hbm, outb.at[pl.ds(0, tail)])
            pltpu.async_copy(
                outb.at[pl.ds(0, tail)],
                out_hbm.at[pl.ds(full * 128, tail)], swb)

        pltpu.make_async_copy(outa, out_hbm.at[pl.ds(0, 128)], swa).wait()

        @pl.when(wid == extra)
        def _():
            pltpu.make_async_copy(
                outb.at[pl.ds(0, tail)],
                out_hbm.at[pl.ds(0, tail)], swb).wait()

        @pl.when(wid != extra)
        def _():
            pltpu.make_async_copy(outb, out_hbm.at[pl.ds(0, 128)], swb).wait()

    def run(tail_vals):
        return k(tt, tail_vals)
    return run


def _sc_gather_add(table, idx3, dense2):
    nw, n_chunks, ch = idx3.shape
    d = table.shape[1]
    qrows = nw * n_chunks * ch // 2
    mesh = plsc.VectorSubcoreMesh(core_axis_name="c", subcore_axis_name="s")

    @functools.partial(
        pl.kernel,
        mesh=mesh,
        out_type=jax.ShapeDtypeStruct((qrows, 2 * d), jnp.float32),
        compiler_params=pltpu.CompilerParams(use_tc_tiling_on_sc=False),
        scratch_types=[
            pltpu.VMEM((n_chunks, ch), jnp.int32),
            pltpu.VMEM((ch, d), jnp.float32),
            pltpu.VMEM((ch, d), jnp.float32),
            pltpu.VMEM((ch // 2, 2 * d), jnp.float32),
            pltpu.VMEM((ch // 2, 2 * d), jnp.float32),
            pltpu.SemaphoreType.DMA,
            pltpu.SemaphoreType.DMA,
            pltpu.SemaphoreType.DMA,
            pltpu.SemaphoreType.DMA,
            pltpu.SemaphoreType.DMA,
            pltpu.SemaphoreType.DMA,
        ],
    )
    def k(table_hbm, idx_hbm, dense_hbm, out_hbm, idx_v, ga, gb, da, db,
          sga, sgb, sda, sdb, swa, swb):
        wid = lax.axis_index("s") * 2 + lax.axis_index("c")
        qb = wid * (n_chunks * ch // 2)
        qc = ch // 2
        pltpu.sync_copy(idx_hbm.at[wid], idx_v)

        def fire(j, gbuf, dbuf, sg, sd):
            pltpu.async_copy(table_hbm.at[idx_v.at[j]], gbuf, sg)
            pltpu.async_copy(dense_hbm.at[pl.ds(qb + j * qc, qc)], dbuf, sd)

        fire(0, ga, da, sga, sda)
        fire(1, gb, db, sgb, sdb)

        def addpack(gbuf, dbuf):
            def row(r, carry):
                q = r >> 1
                base = (r & 1) * d
                for qq in range(4):
                    sl = pl.ds(base + 16 * qq, 16)
                    dbuf[q, sl] = dbuf[q, sl] + gbuf[r, pl.ds(16 * qq, 16)]
                return carry
            lax.fori_loop(0, ch, row, 0)

        def step(g, carry):
            j0 = 2 * g
            j1 = j0 + 1
            pltpu.make_async_copy(table_hbm.at[idx_v.at[0]], ga, sga).wait()
            pltpu.make_async_copy(
                dense_hbm.at[pl.ds(0, qc)], da, sda).wait()
            addpack(ga, da)
            pltpu.async_copy(da, out_hbm.at[pl.ds(qb + j0 * qc, qc)], swa)

            pltpu.make_async_copy(table_hbm.at[idx_v.at[0]], gb, sgb).wait()
            pltpu.make_async_copy(
                dense_hbm.at[pl.ds(0, qc)], db, sdb).wait()
            addpack(gb, db)
            pltpu.async_copy(db, out_hbm.at[pl.ds(qb + j1 * qc, qc)], swb)

            @pl.when(j0 + 2 < n_chunks)
            def _():
                pltpu.make_async_copy(
                    da, out_hbm.at[pl.ds(0, qc)], swa).wait()
                fire(j0 + 2, ga, da, sga, sda)

            @pl.when(j1 + 2 < n_chunks)
            def _():
                pltpu.make_async_copy(
                    db, out_hbm.at[pl.ds(0, qc)], swb).wait()
                fire(j1 + 2, gb, db, sgb, sdb)

            return carry

        lax.fori_loop(0, n_chunks // 2, step, 0)
        pltpu.make_async_copy(da, out_hbm.at[pl.ds(0, qc)], swa).wait()
        pltpu.make_async_copy(db, out_hbm.at[pl.ds(0, qc)], swb).wait()

    return k(table, idx3, dense2)


def _tc_dense(scal8, w0, b0, t2wl, t2bl, tpw0, tpw1m, tpb, valw, valb, d):
    qrows = scal8.shape[0]
    blk = 1024
    grid = qrows // blk

    def half_dense(t, nsf, vp, nvf, w0_, b0_, t2wl_, t2bl_, tpw0_, tpw1m_,
                   tpb_, valw_, valb_):
        lin = t * w0_ + b0_
        s = _psin(t * t2wl_ + t2bl_)
        proj = (lin * tpw0_
                + jnp.dot(s, tpw1m_, preferred_element_type=jnp.float32)
                + tpb_)
        return proj * nsf + (vp * valw_ + valb_ * nvf)

    def body(s8_ref, w0_ref, b0_ref, t2wl_ref, t2bl_ref, tpw0_ref,
             tpw1m_ref, tpb_ref, valw_ref, valb_ref, out_ref):
        s8 = s8_ref[...]
        args = (w0_ref[0, 0], b0_ref[0, 0], t2wl_ref[...], t2bl_ref[...],
                tpw0_ref[...], tpw1m_ref[...], tpb_ref[...], valw_ref[...],
                valb_ref[...])
        even = half_dense(s8[:, 0:1], s8[:, 1:2], s8[:, 2:3], s8[:, 3:4],
                          *args)
        odd = half_dense(s8[:, 4:5], s8[:, 5:6], s8[:, 6:7], s8[:, 7:8],
                         *args)
        out_ref[...] = jnp.concatenate([even, odd], axis=1)

    full = lambda shape: pl.BlockSpec(shape, lambda i: (0, 0))
    row_blk = lambda w: pl.BlockSpec((blk, w), lambda i: (i, 0))
    return pl.pallas_call(
        body,
        grid=(grid,),
        in_specs=[
            row_blk(8),
            full((1, 1)), full((1, 1)), full(t2wl.shape), full(t2bl.shape),
            full(tpw0.shape), full(tpw1m.shape), full(tpb.shape),
            full(valw.shape), full(valb.shape),
        ],
        out_specs=row_blk(2 * d),
        out_shape=jax.ShapeDtypeStruct((qrows, 2 * d), jnp.float32),
    )(scal8, w0, b0, t2wl, t2bl, tpw0, tpw1m, tpb, valw, valb)


def kernel(static_mask, code, numeric_value, time_delta_days,
           numeric_value_mask, table, t2v_w0, t2v_b0, t2v_W, t2v_B,
           tp_W, tp_b, val_W, val_b):
    b, s = code.shape
    d = table.shape[1]
    bs = b * s
    n_chunks = bs // (_NW * _CH)

    idx3 = code.T.astype(jnp.int32).reshape(_NW, n_chunks, _CH)
    nvf = numeric_value_mask.astype(jnp.float32)
    scal8 = jnp.stack([
        time_delta_days.T.reshape(-1),
        (~static_mask).T.reshape(-1).astype(jnp.float32),
        (numeric_value * nvf).T.reshape(-1),
        nvf.T.reshape(-1),
    ], axis=1).reshape(bs // 2, 8)

    tail_rows = (table.shape[0] // 2) % 128
    tail_vals = table[table.shape[0] - 2 * tail_rows:].reshape(tail_rows, 2 * d)
    table_c = _sc_transpose(table.T)(tail_vals).reshape(table.shape[0], d)

    k = t2v_W.shape[0]
    t2wl = jnp.zeros((1, d), jnp.float32).at[0, :k].set(t2v_W)
    t2bl = jnp.zeros((1, d), jnp.float32).at[0, :k].set(t2v_B)
    tpw1m = jnp.zeros((d, d), jnp.float32).at[:k, :].set(tp_W[1:, :])

    dense2 = _tc_dense(
        scal8, t2v_w0.reshape(1, 1), t2v_b0.reshape(1, 1),
        t2wl, t2bl, tp_W[0:1, :], tpw1m, tp_b.reshape(1, -1),
        val_W.reshape(1, -1), val_b.reshape(1, -1), d)

    out2 = _sc_gather_add(table_c, idx3, dense2)
    return out2.reshape(s, b, d).transpose(1, 0, 2)


# speedup vs baseline: 1.5984x; 1.5984x over previous
import functools

import jax
import jax.numpy as jnp
from jax import lax
from jax.experimental import pallas as pl
from jax.experimental.pallas import tpu as pltpu
from jax.experimental.pallas import tpu_sc as plsc

_NW = 32
_CH = 128

_INV2PI = 0.15915494309189535
_TWOPI = 6.283185307179586
_S1 = 9.9998459345e-01
_S3 = -1.6663259377e-01
_S5 = 8.3123882797e-03
_S7 = -1.9316269889e-04
_S9 = 2.1732569601e-06


def _psin(x):
    n = jnp.floor(x * _INV2PI + 0.5)
    r = x - n * _TWOPI
    r2 = r * r
    return r * (_S1 + r2 * (_S3 + r2 * (_S5 + r2 * (_S7 + r2 * _S9))))


def _sc_transpose(tt):
    d, v = tt.shape
    pairs = v // 2
    full = v // 256
    tail = (v - full * 256) // 2
    per_w = full // _NW
    extra = full - per_w * _NW
    mesh = plsc.VectorSubcoreMesh(core_axis_name="c", subcore_axis_name="s")

    @functools.partial(
        pl.kernel,
        mesh=mesh,
        out_type=jax.ShapeDtypeStruct((pairs, 2 * d), jnp.float32),
        compiler_params=pltpu.CompilerParams(use_tc_tiling_on_sc=True,
                                             needs_layout_passes=False),
        scratch_types=[
            pltpu.VMEM((d, 256), jnp.float32),
            pltpu.VMEM((d, 256), jnp.float32),
            pltpu.VMEM((128, 2 * d), jnp.float32),
            pltpu.VMEM((128, 2 * d), jnp.float32),
            pltpu.SemaphoreType.DMA,
            pltpu.SemaphoreType.DMA,
            pltpu.SemaphoreType.DMA,
            pltpu.SemaphoreType.DMA,
        ],
    )
    def k(tt_hbm, tail_hbm, out_hbm, sla, slb, outa, outb, ssa, ssb, swa, swb):
        wid = lax.axis_index("s") * 2 + lax.axis_index("c")
        iota = lax.iota(jnp.int32, 16)
        rowv = [iota + 16 * qq for qq in range(4)]

        def chunk_of(i):
            return i * _NW + wid

        def fire_slab(c, slab, sem):
            pltpu.async_copy(tt_hbm.at[:, pl.ds(c * 256, 256)], slab, sem)

        def transpose_slab(slab, outb_, nu):
            def urow(u, carry):
                for half in range(2):
                    colv = jnp.zeros((16,), jnp.int32) + (2 * u + half)
                    for qq in range(4):
                        vals = plsc.load_gather(slab, [rowv[qq], colv])
                        outb_[u, pl.ds(half * d + 16 * qq, 16)] = vals
                return carry
            lax.fori_loop(0, nu, urow, 0)

        def stage(g, slab, outb_, sem_s, sem_w, off):
            c = chunk_of(2 * g + off)
            pltpu.make_async_copy(tt_hbm.at[:, pl.ds(0, 256)], slab, sem_s).wait()

            @pl.when(g > 0)
            def _():
                pltpu.make_async_copy(outb_, out_hbm.at[pl.ds(0, 128)], sem_w).wait()

            transpose_slab(slab, outb_, 128)
            pltpu.async_copy(outb_, out_hbm.at[pl.ds(c * 128, 128)], sem_w)

            nxt = chunk_of(2 * (g + 1) + off)

            @pl.when(nxt < full)
            def _():
                fire_slab(nxt, slab, sem_s)

        fire_slab(chunk_of(0), sla, ssa)
        fire_slab(chunk_of(1), slb, ssb)

        def body(g, carry):
            stage(g, sla, outa, ssa, swa, 0)
            stage(g, slb, outb, ssb, swb, 1)
            return carry

        lax.fori_loop(0, per_w // 2, body, 0)

        @pl.when(wid < extra)
        def _():
            c = chunk_of(per_w)
            pltpu.make_async_copy(tt_hbm.at[:, pl.ds(0, 256)], sla, ssa).wait()
            pltpu.make_async_copy(outa, out_hbm.at[pl.ds(0, 128)], swa).wait()
            transpose_slab(sla, outa, 128)
            pltpu.async_copy(outa, out_hbm.at[pl.ds(c * 128, 128)], swa)

        @pl.when(wid == extra)
        def _():
            pltpu.make_async_copy(outb, out_hbm.at[pl.ds(0, 128)], swb).wait()
            pltpu.sync_copy(tail_hbm, outb.at[pl.ds(0, tail)])
            pltpu.async_copy(
                outb.at[pl.ds(0, tail)],
                out_hbm.at[pl.ds(full * 128, tail)], swb)

        pltpu.make_async_copy(outa, out_hbm.at[pl.ds(0, 128)], swa).wait()

        @pl.when(wid == extra)
        def _():
            pltpu.make_async_copy(
                outb.at[pl.ds(0, tail)],
                out_hbm.at[pl.ds(0, tail)], swb).wait()

        @pl.when(wid != extra)
        def _():
            pltpu.make_async_copy(outb, out_hbm.at[pl.ds(0, 128)], swb).wait()

    def run(tail_vals):
        return k(tt, tail_vals)
    return run


def _sc_gather_add(table, idx3, dense2):
    nw, n_chunks, ch = idx3.shape
    d = table.shape[1]
    qrows = nw * n_chunks * ch // 2
    mesh = plsc.VectorSubcoreMesh(core_axis_name="c", subcore_axis_name="s")

    @functools.partial(
        pl.kernel,
        mesh=mesh,
        out_type=jax.ShapeDtypeStruct((qrows, 2 * d), jnp.float32),
        compiler_params=pltpu.CompilerParams(use_tc_tiling_on_sc=False),
        scratch_types=[
            pltpu.VMEM((n_chunks, ch), jnp.int32),
            pltpu.VMEM((ch, d), jnp.float32),
            pltpu.VMEM((ch, d), jnp.float32),
            pltpu.VMEM((ch // 2, 2 * d), jnp.float32),
            pltpu.VMEM((ch // 2, 2 * d), jnp.float32),
            pltpu.SemaphoreType.DMA,
            pltpu.SemaphoreType.DMA,
            pltpu.SemaphoreType.DMA,
            pltpu.SemaphoreType.DMA,
            pltpu.SemaphoreType.DMA,
            pltpu.SemaphoreType.DMA,
        ],
    )
    def k(table_hbm, idx_hbm, dense_hbm, out_hbm, idx_v, ga, gb, da, db,
          sga, sgb, sda, sdb, swa, swb):
        wid = lax.axis_index("s") * 2 + lax.axis_index("c")
        qb = wid * (n_chunks * ch // 2)
        qc = ch // 2
        pltpu.sync_copy(idx_hbm.at[wid], idx_v)

        def fire(j, gbuf, dbuf, sg, sd):
            pltpu.async_copy(table_hbm.at[idx_v.at[j]], gbuf, sg)
            pltpu.async_copy(dense_hbm.at[pl.ds(qb + j * qc, qc)], dbuf, sd)

        fire(0, ga, da, sga, sda)
        fire(1, gb, db, sgb, sdb)

        def addpack(gbuf, dbuf):
            def row(r, carry):
                q = r >> 1
                base = (r & 1) * d
                for qq in range(4):
                    sl = pl.ds(base + 16 * qq, 16)
                    dbuf[q, sl] = dbuf[q, sl] + gbuf[r, pl.ds(16 * qq, 16)]
                return carry
            lax.fori_loop(0, ch, row, 0)

        def step(g, carry):
            j0 = 2 * g
            j1 = j0 + 1
            pltpu.make_async_copy(table_hbm.at[idx_v.at[0]], ga, sga).wait()
            pltpu.make_async_copy(
                dense_hbm.at[pl.ds(0, qc)], da, sda).wait()
            addpack(ga, da)
            pltpu.async_copy(da, out_hbm.at[pl.ds(qb + j0 * qc, qc)], swa)

            pltpu.make_async_copy(table_hbm.at[idx_v.at[0]], gb, sgb).wait()
            pltpu.make_async_copy(
                dense_hbm.at[pl.ds(0, qc)], db, sdb).wait()
            addpack(gb, db)
            pltpu.async_copy(db, out_hbm.at[pl.ds(qb + j1 * qc, qc)], swb)

            @pl.when(j0 + 2 < n_chunks)
            def _():
                pltpu.make_async_copy(
                    da, out_hbm.at[pl.ds(0, qc)], swa).wait()
                fire(j0 + 2, ga, da, sga, sda)

            @pl.when(j1 + 2 < n_chunks)
            def _():
                pltpu.make_async_copy(
                    db, out_hbm.at[pl.ds(0, qc)], swb).wait()
                fire(j1 + 2, gb, db, sgb, sdb)

            return carry

        lax.fori_loop(0, n_chunks // 2, step, 0)
        pltpu.make_async_copy(da, out_hbm.at[pl.ds(0, qc)], swa).wait()
        pltpu.make_async_copy(db, out_hbm.at[pl.ds(0, qc)], swb).wait()

    return k(table, idx3, dense2)


def _tc_dense(scal8, w0, b0, t2wl, t2bl, tpw0, tpw1m, tpb, valw, valb, d):
    qrows = scal8.shape[0]
    blk = 1024
    grid = qrows // blk

    def half_dense(t, nsf, vp, nvf, w0_, b0_, t2wl_, t2bl_, tpw0_, tpw1m_,
                   tpb_, valw_, valb_):
        lin = t * w0_ + b0_
        s = _psin(t * t2wl_ + t2bl_)
        proj = (lin * tpw0_
                + jnp.dot(s, tpw1m_, preferred_element_type=jnp.float32)
                + tpb_)
        return proj * nsf + (vp * valw_ + valb_ * nvf)

    def body(s8_ref, w0_ref, b0_ref, t2wl_ref, t2bl_ref, tpw0_ref,
             tpw1m_ref, tpb_ref, valw_ref, valb_ref, out_ref):
        s8 = s8_ref[...]
        args = (w0_ref[0, 0], b0_ref[0, 0], t2wl_ref[...], t2bl_ref[...],
                tpw0_ref[...], tpw1m_ref[...], tpb_ref[...], valw_ref[...],
                valb_ref[...])
        even = half_dense(s8[:, 0:1], s8[:, 1:2], s8[:, 2:3], s8[:, 3:4],
                          *args)
        odd = half_dense(s8[:, 4:5], s8[:, 5:6], s8[:, 6:7], s8[:, 7:8],
                         *args)
        out_ref[...] = jnp.concatenate([even, odd], axis=1)

    full = lambda shape: pl.BlockSpec(shape, lambda i: (0, 0))
    row_blk = lambda w: pl.BlockSpec((blk, w), lambda i: (i, 0))
    return pl.pallas_call(
        body,
        grid=(grid,),
        in_specs=[
            row_blk(8),
            full((1, 1)), full((1, 1)), full(t2wl.shape), full(t2bl.shape),
            full(tpw0.shape), full(tpw1m.shape), full(tpb.shape),
            full(valw.shape), full(valb.shape),
        ],
        out_specs=row_blk(2 * d),
        out_shape=jax.ShapeDtypeStruct((qrows, 2 * d), jnp.float32),
    )(scal8, w0, b0, t2wl, t2bl, tpw0, tpw1m, tpb, valw, valb)


def kernel(static_mask, code, numeric_value, time_delta_days,
           numeric_value_mask, table, t2v_w0, t2v_b0, t2v_W, t2v_B,
           tp_W, tp_b, val_W, val_b):
    b, s = code.shape
    d = table.shape[1]
    bs = b * s
    n_chunks = bs // (_NW * _CH)

    idx3 = code.T.astype(jnp.int32).reshape(_NW, n_chunks, _CH)
    nvf = numeric_value_mask.astype(jnp.float32)
    scal8 = jnp.stack([
        time_delta_days.T.reshape(-1),
        (~static_mask).T.reshape(-1).astype(jnp.float32),
        (numeric_value * nvf).T.reshape(-1),
        nvf.T.reshape(-1),
    ], axis=1).reshape(bs // 2, 8)

    table_c = table

    k = t2v_W.shape[0]
    t2wl = jnp.zeros((1, d), jnp.float32).at[0, :k].set(t2v_W)
    t2bl = jnp.zeros((1, d), jnp.float32).at[0, :k].set(t2v_B)
    tpw1m = jnp.zeros((d, d), jnp.float32).at[:k, :].set(tp_W[1:, :])

    dense2 = _tc_dense(
        scal8, t2v_w0.reshape(1, 1), t2v_b0.reshape(1, 1),
        t2wl, t2bl, tp_W[0:1, :], tpw1m, tp_b.reshape(1, -1),
        val_W.reshape(1, -1), val_b.reshape(1, -1), d)

    out2 = _sc_gather_add(table_c, idx3, dense2)
    return out2.reshape(s, b, d).transpose(1, 0, 2)


# speedup vs baseline: 1.6019x; 1.0022x over previous
import functools

import jax
import jax.numpy as jnp
from jax import lax
from jax.experimental import pallas as pl
from jax.experimental.pallas import tpu as pltpu
from jax.experimental.pallas import tpu_sc as plsc

_NW = 32
_CH = 128

_INV2PI = 0.15915494309189535
_TWOPI = 6.283185307179586
_S1 = 9.9998459345e-01
_S3 = -1.6663259377e-01
_S5 = 8.3123882797e-03
_S7 = -1.9316269889e-04
_S9 = 2.1732569601e-06


def _psin(x):
    n = jnp.floor(x * _INV2PI + 0.5)
    r = x - n * _TWOPI
    r2 = r * r
    return r * (_S1 + r2 * (_S3 + r2 * (_S5 + r2 * (_S7 + r2 * _S9))))


def _sc_gather_add(table, idx3, dense2):
    nw, n_chunks, ch = idx3.shape
    d = table.shape[1]
    qrows = nw * n_chunks * ch // 2
    mesh = plsc.VectorSubcoreMesh(core_axis_name="c", subcore_axis_name="s")

    @functools.partial(
        pl.kernel,
        mesh=mesh,
        out_type=jax.ShapeDtypeStruct((qrows, 2 * d), jnp.float32),
        compiler_params=pltpu.CompilerParams(use_tc_tiling_on_sc=False),
        scratch_types=[
            pltpu.VMEM((n_chunks, ch), jnp.int32),
            pltpu.VMEM((ch, d), jnp.float32),
            pltpu.VMEM((ch, d), jnp.float32),
            pltpu.VMEM((ch // 2, 2 * d), jnp.float32),
            pltpu.VMEM((ch // 2, 2 * d), jnp.float32),
            pltpu.SemaphoreType.DMA,
            pltpu.SemaphoreType.DMA,
            pltpu.SemaphoreType.DMA,
            pltpu.SemaphoreType.DMA,
            pltpu.SemaphoreType.DMA,
            pltpu.SemaphoreType.DMA,
        ],
    )
    def k(table_hbm, idx_hbm, dense_hbm, out_hbm, idx_v, ga, gb, da, db,
          sga, sgb, sda, sdb, swa, swb):
        wid = lax.axis_index("s") * 2 + lax.axis_index("c")
        qb = wid * (n_chunks * ch // 2)
        qc = ch // 2
        pltpu.sync_copy(idx_hbm.at[wid], idx_v)

        def fire(j, gbuf, dbuf, sg, sd):
            pltpu.async_copy(table_hbm.at[idx_v.at[j]], gbuf, sg)
            pltpu.async_copy(dense_hbm.at[pl.ds(qb + j * qc, qc)], dbuf, sd)

        fire(0, ga, da, sga, sda)
        fire(1, gb, db, sgb, sdb)

        def addpack(gbuf, dbuf):
            def row(r, carry):
                q = r >> 1
                base = (r & 1) * d
                for qq in range(4):
                    sl = pl.ds(base + 16 * qq, 16)
                    dbuf[q, sl] = dbuf[q, sl] + gbuf[r, pl.ds(16 * qq, 16)]
                return carry
            lax.fori_loop(0, ch, row, 0)

        def step(g, carry):
            j0 = 2 * g
            j1 = j0 + 1
            pltpu.make_async_copy(table_hbm.at[idx_v.at[0]], ga, sga).wait()
            pltpu.make_async_copy(
                dense_hbm.at[pl.ds(0, qc)], da, sda).wait()
            addpack(ga, da)
            pltpu.async_copy(da, out_hbm.at[pl.ds(qb + j0 * qc, qc)], swa)

            pltpu.make_async_copy(table_hbm.at[idx_v.at[0]], gb, sgb).wait()
            pltpu.make_async_copy(
                dense_hbm.at[pl.ds(0, qc)], db, sdb).wait()
            addpack(gb, db)
            pltpu.async_copy(db, out_hbm.at[pl.ds(qb + j1 * qc, qc)], swb)

            @pl.when(j0 + 2 < n_chunks)
            def _():
                pltpu.make_async_copy(
                    da, out_hbm.at[pl.ds(0, qc)], swa).wait()
                fire(j0 + 2, ga, da, sga, sda)

            @pl.when(j1 + 2 < n_chunks)
            def _():
                pltpu.make_async_copy(
                    db, out_hbm.at[pl.ds(0, qc)], swb).wait()
                fire(j1 + 2, gb, db, sgb, sdb)

            return carry

        lax.fori_loop(0, n_chunks // 2, step, 0)
        pltpu.make_async_copy(da, out_hbm.at[pl.ds(0, qc)], swa).wait()
        pltpu.make_async_copy(db, out_hbm.at[pl.ds(0, qc)], swb).wait()

    return k(table, idx3, dense2)


def _tc_dense(scal8, w0, b0, t2wl, t2bl, tpw0, tpw1m, tpb, valw, valb, d):
    qrows = scal8.shape[0]
    blk = 1024
    grid = qrows // blk

    def half_dense(t, nsf, vp, nvf, w0_, b0_, t2wl_, t2bl_, tpw0_, tpw1m_,
                   tpb_, valw_, valb_):
        lin = t * w0_ + b0_
        s = _psin(t * t2wl_ + t2bl_)
        proj = (lin * tpw0_
                + jnp.dot(s, tpw1m_, preferred_element_type=jnp.float32)
                + tpb_)
        return proj * nsf + (vp * valw_ + valb_ * nvf)

    def body(s8_ref, w0_ref, b0_ref, t2wl_ref, t2bl_ref, tpw0_ref,
             tpw1m_ref, tpb_ref, valw_ref, valb_ref, out_ref):
        s8 = s8_ref[...]
        args = (w0_ref[0, 0], b0_ref[0, 0], t2wl_ref[...], t2bl_ref[...],
                tpw0_ref[...], tpw1m_ref[...], tpb_ref[...], valw_ref[...],
                valb_ref[...])
        even = half_dense(s8[:, 0:1], s8[:, 1:2], s8[:, 2:3], s8[:, 3:4],
                          *args)
        odd = half_dense(s8[:, 4:5], s8[:, 5:6], s8[:, 6:7], s8[:, 7:8],
                         *args)
        out_ref[...] = jnp.concatenate([even, odd], axis=1)

    full = lambda shape: pl.BlockSpec(shape, lambda i: (0, 0))
    row_blk = lambda w: pl.BlockSpec((blk, w), lambda i: (i, 0))
    return pl.pallas_call(
        body,
        grid=(grid,),
        in_specs=[
            row_blk(8),
            full((1, 1)), full((1, 1)), full(t2wl.shape), full(t2bl.shape),
            full(tpw0.shape), full(tpw1m.shape), full(tpb.shape),
            full(valw.shape), full(valb.shape),
        ],
        out_specs=row_blk(2 * d),
        out_shape=jax.ShapeDtypeStruct((qrows, 2 * d), jnp.float32),
    )(scal8, w0, b0, t2wl, t2bl, tpw0, tpw1m, tpb, valw, valb)


def kernel(static_mask, code, numeric_value, time_delta_days,
           numeric_value_mask, table, t2v_w0, t2v_b0, t2v_W, t2v_B,
           tp_W, tp_b, val_W, val_b):
    b, s = code.shape
    d = table.shape[1]
    bs = b * s
    n_chunks = bs // (_NW * _CH)

    idx3 = code.T.astype(jnp.int32).reshape(_NW, n_chunks, _CH)
    nvf = numeric_value_mask.astype(jnp.float32)
    scal8 = jnp.stack([
        time_delta_days.T.reshape(-1),
        (~static_mask).T.reshape(-1).astype(jnp.float32),
        (numeric_value * nvf).T.reshape(-1),
        nvf.T.reshape(-1),
    ], axis=1).reshape(bs // 2, 8)

    table_c = table

    k = t2v_W.shape[0]
    t2wl = jnp.zeros((1, d), jnp.float32).at[0, :k].set(t2v_W)
    t2bl = jnp.zeros((1, d), jnp.float32).at[0, :k].set(t2v_B)
    tpw1m = jnp.zeros((d, d), jnp.float32).at[:k, :].set(tp_W[1:, :])

    dense2 = _tc_dense(
        scal8, t2v_w0.reshape(1, 1), t2v_b0.reshape(1, 1),
        t2wl, t2bl, tp_W[0:1, :], tpw1m, tp_b.reshape(1, -1),
        val_W.reshape(1, -1), val_b.reshape(1, -1), d)

    out2 = _sc_gather_add(table_c, idx3, dense2)
    return out2.reshape(s, b, d).transpose(1, 0, 2)


# speedup vs baseline: 1.6775x; 1.0472x over previous
import functools

import jax
import jax.numpy as jnp
from jax import lax
from jax.experimental import pallas as pl
from jax.experimental.pallas import tpu as pltpu
from jax.experimental.pallas import tpu_sc as plsc

_NW = 32
_CH = 128

_INV2PI = 0.15915494309189535
_TWOPI = 6.283185307179586
_S1 = 9.9998459345e-01
_S3 = -1.6663259377e-01
_S5 = 8.3123882797e-03
_S7 = -1.9316269889e-04
_S9 = 2.1732569601e-06


def _psin(x):
    n = jnp.floor(x * _INV2PI + 0.5)
    r = x - n * _TWOPI
    r2 = r * r
    return r * (_S1 + r2 * (_S3 + r2 * (_S5 + r2 * (_S7 + r2 * _S9))))


def _sc_gather_add(table, idx3, dense2):
    nw, n_chunks, ch = idx3.shape
    d = table.shape[1]
    qrows = nw * n_chunks * ch // 2
    mesh = plsc.VectorSubcoreMesh(core_axis_name="c", subcore_axis_name="s")

    @functools.partial(
        pl.kernel,
        mesh=mesh,
        out_type=jax.ShapeDtypeStruct((qrows, 2 * d), jnp.float32),
        compiler_params=pltpu.CompilerParams(use_tc_tiling_on_sc=False),
        scratch_types=[
            pltpu.VMEM((n_chunks, ch), jnp.int32),
            pltpu.VMEM((ch, d), jnp.float32),
            pltpu.VMEM((ch, d), jnp.float32),
            pltpu.VMEM((ch // 2, 2 * d), jnp.float32),
            pltpu.VMEM((ch // 2, 2 * d), jnp.float32),
            pltpu.SemaphoreType.DMA,
            pltpu.SemaphoreType.DMA,
            pltpu.SemaphoreType.DMA,
            pltpu.SemaphoreType.DMA,
            pltpu.SemaphoreType.DMA,
            pltpu.SemaphoreType.DMA,
        ],
    )
    def k(table_hbm, idx_hbm, dense_hbm, out_hbm, idx_v, ga, gb, da, db,
          sga, sgb, sda, sdb, swa, swb):
        wid = lax.axis_index("s") * 2 + lax.axis_index("c")
        qb = wid * (n_chunks * ch // 2)
        qc = ch // 2
        pltpu.sync_copy(idx_hbm.at[wid], idx_v)

        def fire(j, gbuf, dbuf, sg, sd):
            pltpu.async_copy(table_hbm.at[idx_v.at[j]], gbuf, sg)
            pltpu.async_copy(dense_hbm.at[pl.ds(qb + j * qc, qc)], dbuf, sd)

        fire(0, ga, da, sga, sda)
        fire(1, gb, db, sgb, sdb)

        def addpack(gbuf, dbuf):
            def row(r, carry):
                q = r >> 1
                base = (r & 1) * d
                for qq in range(4):
                    sl = pl.ds(base + 16 * qq, 16)
                    dbuf[q, sl] = dbuf[q, sl] + gbuf[r, pl.ds(16 * qq, 16)]
                return carry
            lax.fori_loop(0, ch, row, 0)

        def step(g, carry):
            j0 = 2 * g
            j1 = j0 + 1
            pltpu.make_async_copy(table_hbm.at[idx_v.at[0]], ga, sga).wait()
            pltpu.make_async_copy(
                dense_hbm.at[pl.ds(0, qc)], da, sda).wait()
            addpack(ga, da)
            pltpu.async_copy(da, out_hbm.at[pl.ds(qb + j0 * qc, qc)], swa)

            pltpu.make_async_copy(table_hbm.at[idx_v.at[0]], gb, sgb).wait()
            pltpu.make_async_copy(
                dense_hbm.at[pl.ds(0, qc)], db, sdb).wait()
            addpack(gb, db)
            pltpu.async_copy(db, out_hbm.at[pl.ds(qb + j1 * qc, qc)], swb)

            @pl.when(j0 + 2 < n_chunks)
            def _():
                pltpu.make_async_copy(
                    da, out_hbm.at[pl.ds(0, qc)], swa).wait()
                fire(j0 + 2, ga, da, sga, sda)

            @pl.when(j1 + 2 < n_chunks)
            def _():
                pltpu.make_async_copy(
                    db, out_hbm.at[pl.ds(0, qc)], swb).wait()
                fire(j1 + 2, gb, db, sgb, sdb)

            return carry

        lax.fori_loop(0, n_chunks // 2, step, 0)
        pltpu.make_async_copy(da, out_hbm.at[pl.ds(0, qc)], swa).wait()
        pltpu.make_async_copy(db, out_hbm.at[pl.ds(0, qc)], swb).wait()

    return k(table, idx3, dense2)


def _tc_dense(scal8, w0, b0, t2wl, t2bl, tpw0, tpw1m, tpb, valw, valb, d):
    qrows = scal8.shape[0]
    blk = 1024
    grid = qrows // blk

    def body(s8_ref, w0_ref, b0_ref, t2wl_ref, t2bl_ref, tpw0_ref,
             tpw1m_ref, tpb_ref, valw_ref, valb_ref, out_ref):
        s8 = s8_ref[...]

        def both(c):
            return jnp.concatenate(
                [jnp.broadcast_to(s8[:, c:c + 1], (blk, d)),
                 jnp.broadcast_to(s8[:, c + 4:c + 5], (blk, d))], axis=1)

        tb, nsfb, vpb, nvfb = both(0), both(1), both(2), both(3)
        lin = tb * w0_ref[0, 0] + b0_ref[0, 0]
        sm = _psin(tb * t2wl_ref[...] + t2bl_ref[...])
        proj = (lin * tpw0_ref[...]
                + jnp.dot(sm, tpw1m_ref[...],
                          preferred_element_type=jnp.float32)
                + tpb_ref[...])
        out_ref[...] = proj * nsfb + (vpb * valw_ref[...]
                                      + valb_ref[...] * nvfb)

    full = lambda shape: pl.BlockSpec(shape, lambda i: (0, 0))
    row_blk = lambda w: pl.BlockSpec((blk, w), lambda i: (i, 0))
    return pl.pallas_call(
        body,
        grid=(grid,),
        in_specs=[
            row_blk(8),
            full((1, 1)), full((1, 1)), full(t2wl.shape), full(t2bl.shape),
            full(tpw0.shape), full(tpw1m.shape), full(tpb.shape),
            full(valw.shape), full(valb.shape),
        ],
        out_specs=row_blk(2 * d),
        out_shape=jax.ShapeDtypeStruct((qrows, 2 * d), jnp.float32),
    )(scal8, w0, b0, t2wl, t2bl, tpw0, tpw1m, tpb, valw, valb)


def kernel(static_mask, code, numeric_value, time_delta_days,
           numeric_value_mask, table, t2v_w0, t2v_b0, t2v_W, t2v_B,
           tp_W, tp_b, val_W, val_b):
    b, s = code.shape
    d = table.shape[1]
    bs = b * s
    n_chunks = bs // (_NW * _CH)

    idx3 = code.T.astype(jnp.int32).reshape(_NW, n_chunks, _CH)
    nvf = numeric_value_mask.astype(jnp.float32)
    scal8 = jnp.stack([
        time_delta_days.T.reshape(-1),
        (~static_mask).T.reshape(-1).astype(jnp.float32),
        (numeric_value * nvf).T.reshape(-1),
        nvf.T.reshape(-1),
    ], axis=1).reshape(bs // 2, 8)

    table_c = table

    k = t2v_W.shape[0]
    tile2 = lambda row: jnp.concatenate([row, row], axis=1)
    t2wl = tile2(jnp.zeros((1, d), jnp.float32).at[0, :k].set(t2v_W))
    t2bl = tile2(jnp.zeros((1, d), jnp.float32).at[0, :k].set(t2v_B))
    m1 = jnp.zeros((d, d), jnp.float32).at[:k, :].set(tp_W[1:, :])
    tpw1m = (jnp.zeros((2 * d, 2 * d), jnp.float32)
             .at[:d, :d].set(m1).at[d:, d:].set(m1))

    dense2 = _tc_dense(
        scal8, t2v_w0.reshape(1, 1), t2v_b0.reshape(1, 1),
        t2wl, t2bl, tile2(tp_W[0:1, :]), tpw1m, tile2(tp_b.reshape(1, -1)),
        tile2(val_W.reshape(1, -1)), tile2(val_b.reshape(1, -1)), d)

    out2 = _sc_gather_add(table_c, idx3, dense2)
    return out2.reshape(s, b, d).transpose(1, 0, 2)


# speedup vs baseline: 1.8224x; 1.0864x over previous
import functools

import jax
import jax.numpy as jnp
from jax import lax
from jax.experimental import pallas as pl
from jax.experimental.pallas import tpu as pltpu
from jax.experimental.pallas import tpu_sc as plsc

_NW = 32
_CH = 128

_INV2PI = 0.15915494309189535
_TWOPI = 6.283185307179586
_S1 = 9.9998459345e-01
_S3 = -1.6663259377e-01
_S5 = 8.3123882797e-03
_S7 = -1.9316269889e-04
_S9 = 2.1732569601e-06


def _psin(x):
    n = jnp.floor(x * _INV2PI + 0.5)
    r = x - n * _TWOPI
    r2 = r * r
    return r * (_S1 + r2 * (_S3 + r2 * (_S5 + r2 * (_S7 + r2 * _S9))))


def _sc_gather_add(table, idx3, dense2):
    nw, n_chunks, ch = idx3.shape
    d = table.shape[1]
    qrows = nw * n_chunks * ch // 2
    mesh = plsc.VectorSubcoreMesh(core_axis_name="c", subcore_axis_name="s")

    @functools.partial(
        pl.kernel,
        mesh=mesh,
        out_type=jax.ShapeDtypeStruct((qrows, 2 * d), jnp.float32),
        compiler_params=pltpu.CompilerParams(use_tc_tiling_on_sc=False),
        scratch_types=[
            pltpu.VMEM((n_chunks, ch), jnp.int32),
            pltpu.VMEM((ch, d), jnp.float32),
            pltpu.VMEM((ch, d), jnp.float32),
            pltpu.VMEM((ch // 2, 2 * d), jnp.float32),
            pltpu.VMEM((ch // 2, 2 * d), jnp.float32),
            pltpu.SemaphoreType.DMA,
            pltpu.SemaphoreType.DMA,
            pltpu.SemaphoreType.DMA,
            pltpu.SemaphoreType.DMA,
            pltpu.SemaphoreType.DMA,
            pltpu.SemaphoreType.DMA,
        ],
    )
    def k(table_hbm, idx_hbm, dense_hbm, out_hbm, idx_v, ga, gb, da, db,
          sga, sgb, sda, sdb, swa, swb):
        wid = lax.axis_index("s") * 2 + lax.axis_index("c")
        qb = wid * (n_chunks * ch // 2)
        qc = ch // 2
        pltpu.sync_copy(idx_hbm.at[wid], idx_v)

        def fire(j, gbuf, dbuf, sg, sd):
            pltpu.async_copy(table_hbm.at[idx_v.at[j]], gbuf, sg)
            pltpu.async_copy(dense_hbm.at[pl.ds(qb + j * qc, qc)], dbuf, sd)

        fire(0, ga, da, sga, sda)
        fire(1, gb, db, sgb, sdb)

        def addpack(gbuf, dbuf):
            def row(q, carry):
                for h in range(2):
                    for qq in range(4):
                        plsc.addupdate(
                            dbuf.at[q, pl.ds(h * d + 16 * qq, 16)],
                            gbuf[2 * q + h, pl.ds(16 * qq, 16)])
                return carry
            lax.fori_loop(0, ch // 2, row, 0)

        def step(g, carry):
            j0 = 2 * g
            j1 = j0 + 1
            pltpu.make_async_copy(table_hbm.at[idx_v.at[0]], ga, sga).wait()
            pltpu.make_async_copy(
                dense_hbm.at[pl.ds(0, qc)], da, sda).wait()
            addpack(ga, da)
            pltpu.async_copy(da, out_hbm.at[pl.ds(qb + j0 * qc, qc)], swa)

            pltpu.make_async_copy(table_hbm.at[idx_v.at[0]], gb, sgb).wait()
            pltpu.make_async_copy(
                dense_hbm.at[pl.ds(0, qc)], db, sdb).wait()
            addpack(gb, db)
            pltpu.async_copy(db, out_hbm.at[pl.ds(qb + j1 * qc, qc)], swb)

            @pl.when(j0 + 2 < n_chunks)
            def _():
                pltpu.make_async_copy(
                    da, out_hbm.at[pl.ds(0, qc)], swa).wait()
                fire(j0 + 2, ga, da, sga, sda)

            @pl.when(j1 + 2 < n_chunks)
            def _():
                pltpu.make_async_copy(
                    db, out_hbm.at[pl.ds(0, qc)], swb).wait()
                fire(j1 + 2, gb, db, sgb, sdb)

            return carry

        lax.fori_loop(0, n_chunks // 2, step, 0)
        pltpu.make_async_copy(da, out_hbm.at[pl.ds(0, qc)], swa).wait()
        pltpu.make_async_copy(db, out_hbm.at[pl.ds(0, qc)], swb).wait()

    return k(table, idx3, dense2)


def _tc_dense(scal8, w0, b0, t2wl, t2bl, tpw0, tpw1m, tpb, valw, valb, d):
    qrows = scal8.shape[0]
    blk = 1024
    grid = qrows // blk

    def body(s8_ref, w0_ref, b0_ref, t2wl_ref, t2bl_ref, tpw0_ref,
             tpw1m_ref, tpb_ref, valw_ref, valb_ref, out_ref):
        s8 = s8_ref[...]

        def both(c):
            return jnp.concatenate(
                [jnp.broadcast_to(s8[:, c:c + 1], (blk, d)),
                 jnp.broadcast_to(s8[:, c + 4:c + 5], (blk, d))], axis=1)

        tb, nsfb, vpb, nvfb = both(0), both(1), both(2), both(3)
        lin = tb * w0_ref[0, 0] + b0_ref[0, 0]
        sm = _psin(tb * t2wl_ref[...] + t2bl_ref[...])
        proj = (lin * tpw0_ref[...]
                + jnp.dot(sm, tpw1m_ref[...],
                          preferred_element_type=jnp.float32)
                + tpb_ref[...])
        out_ref[...] = proj * nsfb + (vpb * valw_ref[...]
                                      + valb_ref[...] * nvfb)

    full = lambda shape: pl.BlockSpec(shape, lambda i: (0, 0))
    row_blk = lambda w: pl.BlockSpec((blk, w), lambda i: (i, 0))
    return pl.pallas_call(
        body,
        grid=(grid,),
        in_specs=[
            row_blk(8),
            full((1, 1)), full((1, 1)), full(t2wl.shape), full(t2bl.shape),
            full(tpw0.shape), full(tpw1m.shape), full(tpb.shape),
            full(valw.shape), full(valb.shape),
        ],
        out_specs=row_blk(2 * d),
        out_shape=jax.ShapeDtypeStruct((qrows, 2 * d), jnp.float32),
    )(scal8, w0, b0, t2wl, t2bl, tpw0, tpw1m, tpb, valw, valb)


def kernel(static_mask, code, numeric_value, time_delta_days,
           numeric_value_mask, table, t2v_w0, t2v_b0, t2v_W, t2v_B,
           tp_W, tp_b, val_W, val_b):
    b, s = code.shape
    d = table.shape[1]
    bs = b * s
    n_chunks = bs // (_NW * _CH)

    idx3 = code.T.astype(jnp.int32).reshape(_NW, n_chunks, _CH)
    nvf = numeric_value_mask.astype(jnp.float32)
    scal8 = jnp.stack([
        time_delta_days.T.reshape(-1),
        (~static_mask).T.reshape(-1).astype(jnp.float32),
        (numeric_value * nvf).T.reshape(-1),
        nvf.T.reshape(-1),
    ], axis=1).reshape(bs // 2, 8)

    table_c = table

    k = t2v_W.shape[0]
    tile2 = lambda row: jnp.concatenate([row, row], axis=1)
    t2wl = tile2(jnp.zeros((1, d), jnp.float32).at[0, :k].set(t2v_W))
    t2bl = tile2(jnp.zeros((1, d), jnp.float32).at[0, :k].set(t2v_B))
    m1 = jnp.zeros((d, d), jnp.float32).at[:k, :].set(tp_W[1:, :])
    tpw1m = (jnp.zeros((2 * d, 2 * d), jnp.float32)
             .at[:d, :d].set(m1).at[d:, d:].set(m1))

    dense2 = _tc_dense(
        scal8, t2v_w0.reshape(1, 1), t2v_b0.reshape(1, 1),
        t2wl, t2bl, tile2(tp_W[0:1, :]), tpw1m, tile2(tp_b.reshape(1, -1)),
        tile2(val_W.reshape(1, -1)), tile2(val_b.reshape(1, -1)), d)

    out2 = _sc_gather_add(table_c, idx3, dense2)
    return out2.reshape(s, b, d).transpose(1, 0, 2)
